# Initial kernel scaffold; baseline (speedup 1.0000x reference)
import jax
import jax.numpy as jnp
from jax.experimental import pallas as pl


def kernel(x, edge_index, node_to_subgraph, subgraph_to_graph, W1_0, b1_0, W2_0, b2_0, W1_1, b1_1, W2_1, b2_1, W1_2, b1_2, W2_2, b2_2, Wh, bh, Wr, br, Wv, bv):
    raise NotImplementedError("write your pallas kernel here")


# speedup vs baseline: 4.3739x; 4.3739x over previous
import jax
import jax.numpy as jnp
from jax.experimental import pallas as pl
from jax.experimental.pallas import tpu as pltpu
from jax.experimental.pallas import tpu_sc as plsc

N = 10000
E = 320000
D = 128
SUB = 1000
G = 64

NC = 2
NS = 16
NW = NC * NS
CHUNK = 128
CH = (E + NW * CHUNK - 1) // (NW * CHUNK)
E_PAD = NW * CH * CHUNK
N_PAD = 10240
RPT = N_PAD // NS

BN = 1000
NBLK = N // BN


def _segsum_body(h_hbm, src_hbm, dst_hbm, zeros_hbm, out_hbm,
                 idx_s, idx_d, rows, acc, sem):
    c = jax.lax.axis_index("c")
    s = jax.lax.axis_index("s")
    wid = c * NS + s
    pltpu.sync_copy(src_hbm.at[wid], idx_s)
    pltpu.sync_copy(dst_hbm.at[wid], idx_d)
    r0 = s * RPT
    pltpu.sync_copy(zeros_hbm, acc.at[pl.ds(r0, RPT)])
    plsc.subcore_barrier()

    def _chunk(j, carry):
        pltpu.async_copy(h_hbm.at[idx_s.at[j]], rows, sem).wait()
        pltpu.sync_copy(rows, acc.at[idx_d.at[j]], add=True)
        return carry

    jax.lax.fori_loop(0, CH, _chunk, 0)
    plsc.subcore_barrier()
    pltpu.sync_copy(acc.at[pl.ds(r0, RPT)], out_hbm.at[c, pl.ds(r0, RPT)])


_SEGSUM_CACHE = []


def _segsum(h, src_p, dst_p, zeros):
    if not _SEGSUM_CACHE:
        _SEGSUM_CACHE.append(pl.kernel(
            _segsum_body,
            out_type=jax.ShapeDtypeStruct((NC, N_PAD, D), jnp.float32),
            mesh=plsc.VectorSubcoreMesh(
                core_axis_name="c", subcore_axis_name="s"),
            scratch_types=[
                pltpu.VMEM((CH, CHUNK), jnp.int32),
                pltpu.VMEM((CH, CHUNK), jnp.int32),
                pltpu.VMEM((CHUNK, D), jnp.float32),
                pltpu.VMEM_SHARED((N_PAD, D), jnp.float32),
                pltpu.SemaphoreType.DMA,
            ],
        ))
    return _SEGSUM_CACHE[0](h, src_p, dst_p, zeros)


def _mlp_body(h_ref, agg_ref, w1_ref, b1_ref, w2_ref, b2_ref, o_ref):
    z = h_ref[...] + agg_ref[0] + agg_ref[1]
    y = jnp.maximum(
        jnp.dot(z, w1_ref[...], preferred_element_type=jnp.float32)
        + b1_ref[...], 0.0)
    o_ref[...] = (jnp.dot(y, w2_ref[...], preferred_element_type=jnp.float32)
                  + b2_ref[...])


def _mlp(h, agg, w1, b1, w2, b2):
    return pl.pallas_call(
        _mlp_body,
        grid=(NBLK,),
        in_specs=[
            pl.BlockSpec((BN, D), lambda i: (i, 0)),
            pl.BlockSpec((NC, BN, D), lambda i: (0, i, 0)),
            pl.BlockSpec((D, D), lambda i: (0, 0)),
            pl.BlockSpec((1, D), lambda i: (0, 0)),
            pl.BlockSpec((D, D), lambda i: (0, 0)),
            pl.BlockSpec((1, D), lambda i: (0, 0)),
        ],
        out_specs=pl.BlockSpec((BN, D), lambda i: (i, 0)),
        out_shape=jax.ShapeDtypeStruct((N, D), jnp.float32),
    )(h, agg, w1, b1, w2, b2)


def _pool_body(h_ref, agg_ref, w1_ref, b1_ref, w2_ref, b2_ref,
               n2s_ref, s2g_ref, wh_ref, bh_ref, wr_ref, br_ref,
               wv_ref, bv_ref, out_ref, var_ref, g_acc):
    i = pl.program_id(0)
    z = h_ref[...] + agg_ref[0] + agg_ref[1]
    y = jnp.maximum(
        jnp.dot(z, w1_ref[...], preferred_element_type=jnp.float32)
        + b1_ref[...], 0.0)
    h3 = (jnp.dot(y, w2_ref[...], preferred_element_type=jnp.float32)
          + b2_ref[...])

    n2s = n2s_ref[0, 0, :]
    s2g = s2g_ref[0, :]
    oh_ns = (n2s[:, None]
             == jax.lax.broadcasted_iota(jnp.int32, (BN, SUB), 1)
             ).astype(jnp.float32)
    oh_sg = (s2g[:, None]
             == jax.lax.broadcasted_iota(jnp.int32, (SUB, G), 1)
             ).astype(jnp.float32)
    oh_ng = jnp.dot(oh_ns, oh_sg, preferred_element_type=jnp.float32)
    contrib = jax.lax.dot_general(
        oh_ng, h3, (((0,), (0,)), ((), ())),
        preferred_element_type=jnp.float32)

    @pl.when(i == 0)
    def _():
        g_acc[...] = jnp.zeros_like(g_acc)

    g_acc[...] += contrib

    @pl.when(i == pl.num_programs(0) - 1)
    def _():
        g = g_acc[...]
        hid = jnp.maximum(
            jnp.dot(g, wh_ref[...], preferred_element_type=jnp.float32)
            + bh_ref[...], 0.0)
        out_ref[...] = (jnp.dot(hid, wr_ref[...],
                                preferred_element_type=jnp.float32)
                        + br_ref[...])
        var_ref[...] = (jnp.dot(hid, wv_ref[...],
                                preferred_element_type=jnp.float32)
                        + bv_ref[...])


def _pool(h, agg, w1, b1, w2, b2, n2s, s2g, wh, bh, wr, br, wv, bv):
    return pl.pallas_call(
        _pool_body,
        grid=(NBLK,),
        in_specs=[
            pl.BlockSpec((BN, D), lambda i: (i, 0)),
            pl.BlockSpec((NC, BN, D), lambda i: (0, i, 0)),
            pl.BlockSpec((D, D), lambda i: (0, 0)),
            pl.BlockSpec((1, D), lambda i: (0, 0)),
            pl.BlockSpec((D, D), lambda i: (0, 0)),
            pl.BlockSpec((1, D), lambda i: (0, 0)),
            pl.BlockSpec((1, 1, BN), lambda i: (i, 0, 0)),
            pl.BlockSpec((1, SUB), lambda i: (0, 0)),
            pl.BlockSpec((D, D), lambda i: (0, 0)),
            pl.BlockSpec((1, D), lambda i: (0, 0)),
            pl.BlockSpec((D, 1), lambda i: (0, 0)),
            pl.BlockSpec((1, 1), lambda i: (0, 0)),
            pl.BlockSpec((D, 1), lambda i: (0, 0)),
            pl.BlockSpec((1, 1), lambda i: (0, 0)),
        ],
        out_specs=[
            pl.BlockSpec((G, 1), lambda i: (0, 0)),
            pl.BlockSpec((G, 1), lambda i: (0, 0)),
        ],
        out_shape=[
            jax.ShapeDtypeStruct((G, 1), jnp.float32),
            jax.ShapeDtypeStruct((G, 1), jnp.float32),
        ],
        scratch_shapes=[pltpu.VMEM((G, D), jnp.float32)],
    )(h, agg, w1, b1, w2, b2, n2s, s2g, wh, bh, wr, br, wv, bv)


def kernel(x, edge_index, node_to_subgraph, subgraph_to_graph,
           W1_0, b1_0, W2_0, b2_0,
           W1_1, b1_1, W2_1, b2_1,
           W1_2, b1_2, W2_2, b2_2,
           Wh, bh, Wr, br, Wv, bv):
    src = edge_index[0]
    dst = edge_index[1]
    pad = E_PAD - E
    src_p = jnp.concatenate(
        [src, jnp.zeros((pad,), jnp.int32)]).reshape(NW, CH, CHUNK)
    dst_p = jnp.concatenate(
        [dst, jnp.full((pad,), N, jnp.int32)]).reshape(NW, CH, CHUNK)
    zeros = jnp.zeros((RPT, D), jnp.float32)

    n2s = node_to_subgraph.reshape(NBLK, 1, BN)
    s2g = subgraph_to_graph.reshape(1, SUB)

    h = x
    for (w1, b1, w2, b2) in ((W1_0, b1_0, W2_0, b2_0),
                             (W1_1, b1_1, W2_1, b2_1)):
        agg = _segsum(h, src_p, dst_p, zeros)
        h = _mlp(h, agg, w1, b1.reshape(1, D), w2, b2.reshape(1, D))

    agg = _segsum(h, src_p, dst_p, zeros)
    out, var = _pool(h, agg,
                     W1_2, b1_2.reshape(1, D), W2_2, b2_2.reshape(1, D),
                     n2s, s2g,
                     Wh, bh.reshape(1, D),
                     Wr, br.reshape(1, 1),
                     Wv, bv.reshape(1, 1))
    return (out, var)
